# baseline (speedup 1.0000x reference)
import functools

import jax
import jax.numpy as jnp
from jax import lax
from jax.experimental import pallas as pl
from jax.experimental.pallas import tpu as pltpu
from jax.experimental.pallas import tpu_sc as plsc

B = 16384
D = 128
NC = 2
NS = 16
NW = NC * NS
RPW = B // NW
CH = 64
NCHUNK = RPW // CH


def _rsqrt_newton(x):
    i = plsc.bitcast(x, jnp.int32)
    i = jnp.int32(0x5F3759DF) - (i >> 1)
    y = plsc.bitcast(i, jnp.float32)
    for _ in range(3):
        y = y * (1.5 - 0.5 * x * y * y)
    return y


def _make_kernel():
    mesh = plsc.VectorSubcoreMesh(core_axis_name="c", subcore_axis_name="s")

    @functools.partial(
        pl.kernel,
        mesh=mesh,
        out_type=jax.ShapeDtypeStruct((NW, 16), jnp.float32),
        compiler_params=pltpu.CompilerParams(needs_layout_passes=False),
        scratch_types=[
            pltpu.VMEM((NCHUNK, CH), jnp.int32),
            pltpu.VMEM((CH, D), jnp.float32),
            pltpu.VMEM((CH, D), jnp.float32),
            pltpu.VMEM((CH, D), jnp.float32),
            pltpu.VMEM((CH, D), jnp.float32),
            pltpu.VMEM((16,), jnp.float32),
            pltpu.SemaphoreType.DMA,
            pltpu.SemaphoreType.DMA,
            pltpu.SemaphoreType.DMA,
            pltpu.SemaphoreType.DMA,
        ],
    )
    def center_loss(feat_h, lab_h, cen_h, out_h,
                    idx_all, f0, f1, r0, r1, acc_v,
                    sg0, sg1, sf0, sf1):
        wid = lax.axis_index("s") * NC + lax.axis_index("c")
        base = wid * RPW
        zero16 = jnp.zeros((16,), jnp.float32)
        lane = lax.iota(jnp.int32, 16)
        dists = (1, 2, 4, 8)
        perms = [lane ^ d for d in dists]
        masks = [(lane & d) != 0 for d in dists]

        dnums = lax.GatherDimensionNumbers(
            offset_dims=(), collapsed_slice_dims=(0,), start_index_map=(0,))

        def vperm(x, p):
            return lax.gather(
                x, p[:, None], dimension_numbers=dnums, slice_sizes=(1,),
                mode=lax.GatherScatterMode.PROMISE_IN_BOUNDS)

        def merge(x, y, lv):
            p = perms[lv]
            return jnp.where(masks[lv], y + vperm(y, p), x + vperm(x, p))

        def start(ci, feat_v, rows_v, sg, sf):
            rbase = base + ci * CH
            pltpu.async_copy(cen_h.at[idx_all.at[ci]], rows_v, sg)
            pltpu.async_copy(feat_h.at[pl.ds(rbase, CH)], feat_v, sf)

        def wait_chunk(feat_v, rows_v, sg, sf):
            pltpu.make_async_copy(cen_h.at[idx_all.at[0]], rows_v, sg).wait()
            pltpu.make_async_copy(feat_h.at[pl.ds(base, CH)], feat_v, sf).wait()

        def compute(feat_v, rows_v, loss16, cc16):
            def grp(gi, carry):
                l16, cc = carry
                row0 = gi * 16
                ff16 = zero16
                fc16 = zero16
                for p in range(16):
                    rr = row0 + p
                    ff = None
                    fc = None
                    for k in range(D // 16):
                        fv = feat_v[rr, pl.ds(k * 16, 16)]
                        cv = rows_v[rr, pl.ds(k * 16, 16)]
                        ff = fv * fv if ff is None else ff + fv * fv
                        fc = fv * cv if fc is None else fc + fv * cv
                        cc = cc + cv * cv
                    sel = lane == p
                    ff16 = jnp.where(sel, jnp.sum(ff), ff16)
                    fc16 = jnp.where(sel, jnp.sum(fc), fc16)
                inv = jnp.minimum(_rsqrt_newton(ff16), 1e12)
                return l16 + ff16 * inv * inv - 2.0 * fc16 * inv, cc

            return lax.fori_loop(0, CH // 16, grp, (loss16, cc16),
                                 unroll=2)

        pltpu.sync_copy(lab_h.at[pl.ds(wid * NCHUNK, NCHUNK)], idx_all)
        start(0, f0, r0, sg0, sf0)

        def pair(j, carry):
            l16, cc = carry
            start(2 * j + 1, f1, r1, sg1, sf1)
            wait_chunk(f0, r0, sg0, sf0)
            l16, cc = compute(f0, r0, l16, cc)

            @pl.when(j < (NCHUNK // 2) - 1)
            def _():
                start(2 * j + 2, f0, r0, sg0, sf0)

            wait_chunk(f1, r1, sg1, sf1)
            l16, cc = compute(f1, r1, l16, cc)
            return l16, cc

        loss16, cc16 = lax.fori_loop(0, NCHUNK // 2, pair, (zero16, zero16))
        acc_v[...] = (loss16 + cc16) * (0.5 / B)
        pltpu.sync_copy(acc_v, out_h.at[wid])

    return center_loss


_center_loss = _make_kernel()


@jax.jit
def kernel(features, labels, centers):
    lab2 = labels.astype(jnp.int32).reshape(B // CH, CH)
    partials = _center_loss(features, lab2, centers)
    return jnp.sum(partials)


# speedup vs baseline: 1.0408x; 1.0408x over previous
import functools

import jax
import jax.numpy as jnp
from jax import lax
from jax.experimental import pallas as pl
from jax.experimental.pallas import tpu as pltpu
from jax.experimental.pallas import tpu_sc as plsc

B = 16384
D = 128
NC = 2
NS = 16
NW = NC * NS
RPW = B // NW
CH = 64
NCHUNK = RPW // CH


def _rsqrt_newton(x):
    i = plsc.bitcast(x, jnp.int32)
    i = jnp.int32(0x5F3759DF) - (i >> 1)
    y = plsc.bitcast(i, jnp.float32)
    for _ in range(3):
        y = y * (1.5 - 0.5 * x * y * y)
    return y


def _make_kernel():
    mesh = plsc.VectorSubcoreMesh(core_axis_name="c", subcore_axis_name="s")

    @functools.partial(
        pl.kernel,
        mesh=mesh,
        out_type=jax.ShapeDtypeStruct((NW, 16), jnp.float32),
        compiler_params=pltpu.CompilerParams(needs_layout_passes=False),
        scratch_types=[
            pltpu.VMEM((NCHUNK, CH), jnp.int32),
            pltpu.VMEM((CH, D), jnp.float32),
            pltpu.VMEM((CH, D), jnp.float32),
            pltpu.VMEM((CH, D), jnp.float32),
            pltpu.VMEM((CH, D), jnp.float32),
            pltpu.VMEM((16,), jnp.float32),
            pltpu.SemaphoreType.DMA,
            pltpu.SemaphoreType.DMA,
            pltpu.SemaphoreType.DMA,
            pltpu.SemaphoreType.DMA,
        ],
    )
    def center_loss(feat_h, lab_h, cen_h, out_h,
                    idx_all, f0, f1, r0, r1, acc_v,
                    sg0, sg1, sf0, sf1):
        wid = lax.axis_index("s") * NC + lax.axis_index("c")
        base = wid * RPW
        zero16 = jnp.zeros((16,), jnp.float32)
        lane = lax.iota(jnp.int32, 16)
        dists = (1, 2, 4, 8)
        perms = [lane ^ d for d in dists]
        masks = [(lane & d) != 0 for d in dists]

        dnums = lax.GatherDimensionNumbers(
            offset_dims=(), collapsed_slice_dims=(0,), start_index_map=(0,))

        def vperm(x, p):
            return lax.gather(
                x, p[:, None], dimension_numbers=dnums, slice_sizes=(1,),
                mode=lax.GatherScatterMode.PROMISE_IN_BOUNDS)

        def merge(x, y, lv):
            p = perms[lv]
            return jnp.where(masks[lv], y + vperm(y, p), x + vperm(x, p))

        def start(ci, feat_v, rows_v, sg, sf):
            rbase = base + ci * CH
            pltpu.async_copy(cen_h.at[idx_all.at[ci]], rows_v, sg)
            pltpu.async_copy(feat_h.at[pl.ds(rbase, CH)], feat_v, sf)

        def wait_chunk(feat_v, rows_v, sg, sf):
            pltpu.make_async_copy(cen_h.at[idx_all.at[0]], rows_v, sg).wait()
            pltpu.make_async_copy(feat_h.at[pl.ds(base, CH)], feat_v, sf).wait()

        def compute(feat_v, rows_v, loss16, cc16):
            def grp(gi, carry):
                l16, cc = carry
                row0 = gi * 16
                ff16 = zero16
                fc16 = zero16
                for p in range(16):
                    rr = row0 + p
                    ff = None
                    fc = None
                    for k in range(D // 16):
                        fv = feat_v[rr, pl.ds(k * 16, 16)]
                        cv = rows_v[rr, pl.ds(k * 16, 16)]
                        ff = fv * fv if ff is None else ff + fv * fv
                        fc = fv * cv if fc is None else fc + fv * cv
                        cc = cc + cv * cv
                    sel = lane == p
                    ff16 = jnp.where(sel, jnp.sum(ff), ff16)
                    fc16 = jnp.where(sel, jnp.sum(fc), fc16)
                inv = jnp.minimum(_rsqrt_newton(ff16), 1e12)
                return l16 + ff16 * inv * inv - 2.0 * fc16 * inv, cc

            return lax.fori_loop(0, CH // 16, grp, (loss16, cc16))

        pltpu.sync_copy(lab_h.at[pl.ds(wid * NCHUNK, NCHUNK)], idx_all)
        start(0, f0, r0, sg0, sf0)

        def pair(j, carry):
            l16, cc = carry
            start(2 * j + 1, f1, r1, sg1, sf1)
            wait_chunk(f0, r0, sg0, sf0)
            l16, cc = compute(f0, r0, l16, cc)

            @pl.when(j < (NCHUNK // 2) - 1)
            def _():
                start(2 * j + 2, f0, r0, sg0, sf0)

            wait_chunk(f1, r1, sg1, sf1)
            l16, cc = compute(f1, r1, l16, cc)
            return l16, cc

        loss16, cc16 = lax.fori_loop(0, NCHUNK // 2, pair, (zero16, zero16))
        acc_v[...] = (loss16 + cc16) * (0.5 / B)
        pltpu.sync_copy(acc_v, out_h.at[wid])

    return center_loss


_center_loss = _make_kernel()


@jax.jit
def kernel(features, labels, centers):
    lab2 = labels.astype(jnp.int32).reshape(B // CH, CH)
    partials = _center_loss(features, lab2, centers)
    return jnp.sum(partials)
